# Initial kernel scaffold; baseline (speedup 1.0000x reference)
import jax
import jax.numpy as jnp
from jax.experimental import pallas as pl


def kernel(x, edge_index, conv_w, conv_b, lin_w, lin_b, gat_w, att_src, att_dst, gat_bias):
    raise NotImplementedError("write your pallas kernel here")


# speedup vs baseline: 1.0474x; 1.0474x over previous
import functools

import jax
import jax.numpy as jnp
from jax.experimental import pallas as pl
from jax.experimental.pallas import tpu as pltpu

N_NODES = 10000
HIDDEN = 40
HPAD = 48
NCHUNK = 4
CW = 104
PC = 96


def _cnn_body(x_ref, we_ref, wo_ref, cb_ref, lw_ref, lb_ref, gw_ref,
              asv_ref, adv_ref, out_ref):
    M = x_ref.shape[0]
    acc = jnp.broadcast_to(lb_ref[...][None, :], (M, 80))
    for k in range(NCHUNK):
        xk = x_ref[:, 96 * k: 96 * k + CW]
        ye = xk @ we_ref[k] + cb_ref[...][None, :]
        yo = xk @ wo_ref[k] + cb_ref[...][None, :]
        zk = jnp.maximum(jnp.maximum(ye, yo), 0.0)
        acc = acc + zk @ lw_ref[k]
    feat = jnp.maximum(acc, 0.0)
    hl = feat @ gw_ref[...]
    a_s = jnp.sum(hl * asv_ref[...][None, :], axis=-1, keepdims=True)
    a_d = jnp.sum(hl * adv_ref[...][None, :], axis=-1, keepdims=True)
    out_ref[:, 0:HIDDEN] = hl
    out_ref[:, HIDDEN:HIDDEN + 1] = a_s
    out_ref[:, HIDDEN + 1:HIDDEN + 2] = a_d
    out_ref[:, HIDDEN + 2:HPAD] = jnp.zeros((M, HPAD - HIDDEN - 2),
                                            dtype=jnp.float32)


def _make_band_weights(conv_w):
    w = conv_w[:, 0, :]
    i = jnp.arange(392)[:, None]
    q = jnp.arange(48)[None, :]
    cols_c = jnp.arange(8)[None, :]

    def band(off):
        j = i - 8 * q - off
        valid = (j >= 0) & (j < 8)
        jc = jnp.clip(j, 0, 7)
        taps = w.T[jc]
        taps = jnp.where(valid[:, :, None], taps, 0.0)
        return taps.reshape(392, 48 * 8)

    return band(0), band(4)


def _cnn_stage(xs2, we, wo, cb_cols, lw_perm, lin_b, gat_w, att_src, att_dst):
    M = xs2.shape[0]
    B = 2000
    grid = (M // B,)
    full = lambda *s: pl.BlockSpec(s, lambda i: tuple(0 for _ in s))
    return pl.pallas_call(
        _cnn_body,
        grid=grid,
        in_specs=[
            pl.BlockSpec((B, 392), lambda i: (i, 0)),
            full(NCHUNK, CW, PC), full(NCHUNK, CW, PC), full(PC,),
            full(NCHUNK, PC, 80), full(80,), full(80, HIDDEN),
            full(HIDDEN,), full(HIDDEN,),
        ],
        out_specs=pl.BlockSpec((B, HPAD), lambda i: (i, 0)),
        out_shape=jax.ShapeDtypeStruct((M, HPAD), jnp.float32),
    )(xs2, we, wo, cb_cols, lw_perm, lin_b, gat_w, att_src, att_dst)


def kernel(x, edge_index, conv_w, conv_b, lin_w, lin_b, gat_w, att_src,
           att_dst, gat_bias):
    n = x.shape[0]
    xs2 = x[:, :, 3:].reshape(n * 5, 392)
    w_even, w_odd = _make_band_weights(conv_w)
    we = jnp.stack([jax.lax.dynamic_slice(w_even, (96 * k, 96 * k), (CW, PC))
                    for k in range(NCHUNK)])
    wo = jnp.stack([jax.lax.dynamic_slice(w_odd, (96 * k, 96 * k), (CW, PC))
                    for k in range(NCHUNK)])
    cb_cols = jnp.tile(conv_b, PC // 8)
    q = jnp.arange(48)
    c = jnp.arange(8)
    rows = (c[None, :] * 48 + q[:, None]).reshape(-1)
    lw_perm = lin_w[rows].reshape(NCHUNK, PC, 80)

    out48 = _cnn_stage(xs2, we, wo, cb_cols, lw_perm, lin_b, gat_w,
                       att_src, att_dst)
    per_ns = out48.reshape(n, 5, HPAD)
    hl = jnp.transpose(per_ns[:, :, :HIDDEN], (1, 0, 2))
    a_s = jnp.transpose(per_ns[:, :, HIDDEN], (1, 0))
    a_d = jnp.transpose(per_ns[:, :, HIDDEN + 1], (1, 0))

    src = edge_index[0].astype(jnp.int32)
    dst = edge_index[1].astype(jnp.int32)

    def one(hlc, asc, adc):
        e = asc[src] + adc[dst]
        e = jnp.maximum(e, 0.2 * e)
        ex = jnp.exp(e)
        denom = jax.ops.segment_sum(ex, dst, num_segments=n)
        alpha = ex / (denom[dst] + 1e-16)
        msg = hlc[src] * alpha[:, None]
        out = jax.ops.segment_max(msg, dst, num_segments=n)
        out = jnp.where(jnp.isfinite(out), out, 0.0)
        return out + gat_bias[None, :]

    per = jax.vmap(one)(hl, a_s, a_d)
    return jax.nn.relu(per)


# speedup vs baseline: 22.5200x; 21.5012x over previous
import functools

import jax
import jax.numpy as jnp
from jax import lax
from jax.experimental import pallas as pl
from jax.experimental.pallas import tpu as pltpu
from jax.experimental.pallas import tpu_sc as plsc

N_NODES = 10000
HIDDEN = 40
HPAD = 48
NP = 10240
NODES_W = 320
NW = 32
E = 640000
ECHUNK = 4000
QCAP = 23040
GB = 128
NEG = -3.0e38
NCHUNK = 4
CW = 104
PC = 96


def _cnn_body(x_ref, we_ref, wo_ref, cb_ref, lw_ref, lb_ref, gw_ref,
              asv_ref, adv_ref, out_ref):
    M = x_ref.shape[0]
    acc = jnp.broadcast_to(lb_ref[...][None, :], (M, 80))
    for k in range(NCHUNK):
        xk = x_ref[:, 96 * k: 96 * k + CW]
        ye = xk @ we_ref[k] + cb_ref[...][None, :]
        yo = xk @ wo_ref[k] + cb_ref[...][None, :]
        zk = jnp.maximum(jnp.maximum(ye, yo), 0.0)
        acc = acc + zk @ lw_ref[k]
    feat = jnp.maximum(acc, 0.0)
    hl = feat @ gw_ref[...]
    a_s = jnp.sum(hl * asv_ref[...][None, :], axis=-1, keepdims=True)
    a_d = jnp.sum(hl * adv_ref[...][None, :], axis=-1, keepdims=True)
    out_ref[:, 0:HIDDEN] = hl
    out_ref[:, HIDDEN:HIDDEN + 1] = a_s
    out_ref[:, HIDDEN + 1:HIDDEN + 2] = a_d
    out_ref[:, HIDDEN + 2:HPAD] = jnp.zeros((M, HPAD - HIDDEN - 2),
                                            dtype=jnp.float32)


def _make_band_weights(conv_w):
    w = conv_w[:, 0, :]
    i = jnp.arange(392)[:, None]
    q = jnp.arange(48)[None, :]
    cols_c = jnp.arange(8)[None, :]

    def band(off):
        j = i - 8 * q - off
        valid = (j >= 0) & (j < 8)
        jc = jnp.clip(j, 0, 7)
        taps = w.T[jc]
        taps = jnp.where(valid[:, :, None], taps, 0.0)
        return taps.reshape(392, 48 * 8)

    return band(0), band(4)


def _cnn_stage(xs2, we, wo, cb_cols, lw_perm, lin_b, gat_w, att_src, att_dst):
    M = xs2.shape[0]
    B = 2000
    grid = (M // B,)
    full = lambda *s: pl.BlockSpec(s, lambda i: tuple(0 for _ in s))
    return pl.pallas_call(
        _cnn_body,
        grid=grid,
        in_specs=[
            pl.BlockSpec((B, 392), lambda i: (i, 0)),
            full(NCHUNK, CW, PC), full(NCHUNK, CW, PC), full(PC,),
            full(NCHUNK, PC, 80), full(80,), full(80, HIDDEN),
            full(HIDDEN,), full(HIDDEN,),
        ],
        out_specs=pl.BlockSpec((B, HPAD), lambda i: (i, 0)),
        out_shape=jax.ShapeDtypeStruct((M, HPAD), jnp.float32),
    )(xs2, we, wo, cb_cols, lw_perm, lin_b, gat_w, att_src, att_dst)


def _sc_edge_body(tbl, srcv, dstv, asp, adp, bias48, out,
                  qsrc, qdstl, qex, asv, adw, denw, acc, srcc, dstc,
                  qrow, rows, biasv, sem):
    wid = lax.axis_index("s") * 2 + lax.axis_index("c")
    lo = wid * NODES_W
    lane = lax.broadcasted_iota(jnp.int32, (16,), 0)

    pltpu.sync_copy(bias48, biasv)

    def zi(i, _):
        z = jnp.zeros((16,), jnp.int32)
        qsrc[pl.ds(i * 16, 16)] = z
        qdstl[pl.ds(i * 16, 16)] = z
        return 0
    lax.fori_loop(0, QCAP // 16, zi, 0)

    def scan_chunk(ci, qn):
        off = ci * ECHUNK
        pltpu.sync_copy(srcv.at[pl.ds(off, ECHUNK)], srcc)
        pltpu.sync_copy(dstv.at[pl.ds(off, ECHUNK)], dstc)

        def grp(g, qn):
            vs = srcc[pl.ds(g * 16, 16)]
            vd = dstc[pl.ds(g * 16, 16)]
            inb = (vd >= lo) & (vd < lo + NODES_W)
            plsc.store_compressed(qsrc.at[pl.ds(qn, 16)], vs, mask=inb)
            plsc.store_compressed(qdstl.at[pl.ds(qn, 16)], vd - lo, mask=inb)
            return qn + jnp.sum(inb.astype(jnp.int32))

        return lax.fori_loop(0, ECHUNK // 16, grp, qn)

    qn = lax.fori_loop(0, E // ECHUNK, scan_chunk, jnp.int32(0))
    ngrp = (qn + 15) // 16

    def channel(s, _):
        pltpu.sync_copy(asp.at[pl.ds(s * NP, NP)], asv)
        pltpu.sync_copy(adp.at[pl.ds(s * NP + lo, NODES_W)], adw)

        def zd(i, _):
            denw[pl.ds(i * 16, 16)] = jnp.zeros((16,), jnp.float32)
            return 0
        lax.fori_loop(0, NODES_W // 16, zd, 0)

        def za(i, _):
            acc[pl.ds(i * 16, 16)] = jnp.full((16,), NEG, jnp.float32)
            return 0
        lax.fori_loop(0, (NODES_W + 1) * HPAD // 16, za, 0)

        def p1(g, _):
            base = g * 16
            m = (base + lane) < qn
            vsrc = qsrc[pl.ds(base, 16)]
            vdl = qdstl[pl.ds(base, 16)]
            va = plsc.load_gather(asv, [vsrc])
            vb = plsc.load_gather(adw, [vdl])
            ve = va + vb
            ve = jnp.maximum(ve, 0.2 * ve)
            vex = jnp.exp(ve)
            qex[pl.ds(base, 16)] = vex
            plsc.addupdate_scatter(denw, [vdl], jnp.where(m, vex, 0.0))
            return 0
        lax.fori_loop(0, ngrp, p1, 0)

        def p2(g, _):
            base = g * 16
            vdl = qdstl[pl.ds(base, 16)]
            vden = plsc.load_gather(denw, [vdl])
            qex[pl.ds(base, 16)] = qex[pl.ds(base, 16)] / (vden + 1e-16)
            return 0
        lax.fori_loop(0, ngrp, p2, 0)

        nblk = (qn + GB - 1) // GB

        def blk(b, _):
            qb = b * GB

            def bld(g, _):
                v = qsrc[pl.ds(qb + g * 16, 16)] * 5 + s
                qrow[pl.ds(g * 16, 16)] = v
                return 0
            lax.fori_loop(0, GB // 16, bld, 0)
            pltpu.async_copy(tbl.at[qrow], rows, sem).wait()

            def grp16(gi, _):
                base = qb + gi * 16
                valid = (base + lane) < qn
                alpha16 = qex[pl.ds(base, 16)]
                dl16 = jnp.where(valid, qdstl[pl.ds(base, 16)], NODES_W)
                for i in range(16):
                    a = alpha16[i]
                    ab = dl16[i] * HPAD
                    jj = gi * 16 + i
                    for t in range(3):
                        rv = rows[jj, pl.ds(t * 16, 16)]
                        av = acc[pl.ds(ab + t * 16, 16)]
                        acc[pl.ds(ab + t * 16, 16)] = jnp.maximum(av, a * rv)
                return 0
            lax.fori_loop(0, GB // 16, grp16, 0)
            return 0
        lax.fori_loop(0, nblk, blk, 0)

        def ep(r, _):
            rb = r * HPAD
            for t in range(3):
                a = acc[pl.ds(rb + t * 16, 16)]
                bv = biasv[pl.ds(t * 16, 16)]
                v = jnp.where(a < -1.0e37, 0.0, a) + bv
                acc[pl.ds(rb + t * 16, 16)] = jnp.maximum(v, 0.0)
            return 0
        lax.fori_loop(0, NODES_W, ep, 0)
        pltpu.sync_copy(acc.at[pl.ds(0, NODES_W * HPAD)],
                        out.at[pl.ds((s * NP + lo) * HPAD, NODES_W * HPAD)])
        return 0

    lax.fori_loop(0, 5, channel, 0)


def _sc_edge_stage(tbl, srcv, dstv, asp, adp, bias48):
    mesh = plsc.VectorSubcoreMesh(core_axis_name="c", subcore_axis_name="s")
    f = functools.partial(
        pl.kernel,
        mesh=mesh,
        compiler_params=pltpu.CompilerParams(needs_layout_passes=False,
                                             use_tc_tiling_on_sc=False),
        out_type=jax.ShapeDtypeStruct((5 * NP * HPAD,), jnp.float32),
        scratch_types=[
            pltpu.VMEM((QCAP,), jnp.int32),
            pltpu.VMEM((QCAP,), jnp.int32),
            pltpu.VMEM((QCAP,), jnp.float32),
            pltpu.VMEM((NP,), jnp.float32),
            pltpu.VMEM((NODES_W,), jnp.float32),
            pltpu.VMEM((NODES_W,), jnp.float32),
            pltpu.VMEM(((NODES_W + 1) * HPAD,), jnp.float32),
            pltpu.VMEM((ECHUNK,), jnp.int32),
            pltpu.VMEM((ECHUNK,), jnp.int32),
            pltpu.VMEM((GB,), jnp.int32),
            pltpu.VMEM((GB, HPAD), jnp.float32),
            pltpu.VMEM((HPAD,), jnp.float32),
            pltpu.SemaphoreType.DMA,
        ],
    )(_sc_edge_body)
    return f(tbl, srcv, dstv, asp, adp, bias48)


def kernel(x, edge_index, conv_w, conv_b, lin_w, lin_b, gat_w, att_src,
           att_dst, gat_bias):
    n = x.shape[0]
    xs2 = x[:, :, 3:].reshape(n * 5, 392)
    w_even, w_odd = _make_band_weights(conv_w)
    we = jnp.stack([jax.lax.dynamic_slice(w_even, (96 * k, 96 * k), (CW, PC))
                    for k in range(NCHUNK)])
    wo = jnp.stack([jax.lax.dynamic_slice(w_odd, (96 * k, 96 * k), (CW, PC))
                    for k in range(NCHUNK)])
    cb_cols = jnp.tile(conv_b, PC // 8)
    q = jnp.arange(48)
    c = jnp.arange(8)
    rows = (c[None, :] * 48 + q[:, None]).reshape(-1)
    lw_perm = lin_w[rows].reshape(NCHUNK, PC, 80)

    out48 = _cnn_stage(xs2, we, wo, cb_cols, lw_perm, lin_b, gat_w,
                       att_src, att_dst)

    src = edge_index[0].astype(jnp.int32)
    dst = edge_index[1].astype(jnp.int32)
    asp = jnp.pad(out48[:, HIDDEN].reshape(n, 5).T, ((0, 0), (0, NP - n))).reshape(-1)
    adp = jnp.pad(out48[:, HIDDEN + 1].reshape(n, 5).T, ((0, 0), (0, NP - n))).reshape(-1)
    bias48 = jnp.pad(gat_bias, (0, HPAD - HIDDEN))

    outp = _sc_edge_stage(out48, src, dst, asp, adp, bias48)
    return outp.reshape(5, NP, HPAD)[:, :n, :HIDDEN]


# speedup vs baseline: 22.5230x; 1.0001x over previous
import functools

import jax
import jax.numpy as jnp
from jax import lax
from jax.experimental import pallas as pl
from jax.experimental.pallas import tpu as pltpu
from jax.experimental.pallas import tpu_sc as plsc

N_NODES = 10000
HIDDEN = 40
HPAD = 48
NP = 10240
NODES_W = 320
NW = 32
E = 640000
ECHUNK = 4000
QCAP = 23040
GB = 128
NEG = -3.0e38
NCHUNK = 4
CW = 104
PC = 96


def _cnn_body(x_ref, we_ref, wo_ref, cb_ref, lw_ref, lb_ref, gw_ref,
              asv_ref, adv_ref, out_ref):
    M = x_ref.shape[0]
    acc = jnp.broadcast_to(lb_ref[...][None, :], (M, 80))
    for k in range(NCHUNK):
        xk = x_ref[:, 96 * k: 96 * k + CW]
        ye = xk @ we_ref[k] + cb_ref[...][None, :]
        yo = xk @ wo_ref[k] + cb_ref[...][None, :]
        zk = jnp.maximum(jnp.maximum(ye, yo), 0.0)
        acc = acc + zk @ lw_ref[k]
    feat = jnp.maximum(acc, 0.0)
    hl = feat @ gw_ref[...]
    a_s = jnp.sum(hl * asv_ref[...][None, :], axis=-1, keepdims=True)
    a_d = jnp.sum(hl * adv_ref[...][None, :], axis=-1, keepdims=True)
    out_ref[:, 0:HIDDEN] = hl
    out_ref[:, HIDDEN:HIDDEN + 1] = a_s
    out_ref[:, HIDDEN + 1:HIDDEN + 2] = a_d
    out_ref[:, HIDDEN + 2:HPAD] = jnp.zeros((M, HPAD - HIDDEN - 2),
                                            dtype=jnp.float32)


def _make_band_weights(conv_w):
    w = conv_w[:, 0, :]
    i = jnp.arange(392)[:, None]
    q = jnp.arange(48)[None, :]
    cols_c = jnp.arange(8)[None, :]

    def band(off):
        j = i - 8 * q - off
        valid = (j >= 0) & (j < 8)
        jc = jnp.clip(j, 0, 7)
        taps = w.T[jc]
        taps = jnp.where(valid[:, :, None], taps, 0.0)
        return taps.reshape(392, 48 * 8)

    return band(0), band(4)


def _cnn_stage(xs2, we, wo, cb_cols, lw_perm, lin_b, gat_w, att_src, att_dst):
    M = xs2.shape[0]
    B = 2000
    grid = (M // B,)
    full = lambda *s: pl.BlockSpec(s, lambda i: tuple(0 for _ in s))
    return pl.pallas_call(
        _cnn_body,
        grid=grid,
        in_specs=[
            pl.BlockSpec((B, 392), lambda i: (i, 0)),
            full(NCHUNK, CW, PC), full(NCHUNK, CW, PC), full(PC,),
            full(NCHUNK, PC, 80), full(80,), full(80, HIDDEN),
            full(HIDDEN,), full(HIDDEN,),
        ],
        out_specs=pl.BlockSpec((B, HPAD), lambda i: (i, 0)),
        out_shape=jax.ShapeDtypeStruct((M, HPAD), jnp.float32),
    )(xs2, we, wo, cb_cols, lw_perm, lin_b, gat_w, att_src, att_dst)


def _sc_edge_body(tbl, srcv, dstv, asp, adp, bias48, out,
                  qsrc, qdstl, qex, asv, adw, denw, acc, accb, srcc, dstc,
                  qrow, rows, biasv, sem):
    wid = lax.axis_index("s") * 2 + lax.axis_index("c")
    lo = wid * NODES_W
    lane = lax.broadcasted_iota(jnp.int32, (16,), 0)

    pltpu.sync_copy(bias48, biasv)

    def zi(i, _):
        z = jnp.zeros((16,), jnp.int32)
        qsrc[pl.ds(i * 16, 16)] = z
        qdstl[pl.ds(i * 16, 16)] = z
        return 0
    lax.fori_loop(0, QCAP // 16, zi, 0)

    def scan_chunk(ci, qn):
        off = ci * ECHUNK
        pltpu.sync_copy(srcv.at[pl.ds(off, ECHUNK)], srcc)
        pltpu.sync_copy(dstv.at[pl.ds(off, ECHUNK)], dstc)

        def grp(g, qn):
            vs = srcc[pl.ds(g * 16, 16)]
            vd = dstc[pl.ds(g * 16, 16)]
            vdl = vd - lo
            inb = vdl.astype(jnp.uint32) < jnp.uint32(NODES_W)
            plsc.store_compressed(qsrc.at[pl.ds(qn, 16)], vs, mask=inb)
            plsc.store_compressed(qdstl.at[pl.ds(qn, 16)], vdl, mask=inb)
            return qn + jnp.sum(inb.astype(jnp.int32))

        return lax.fori_loop(0, ECHUNK // 16, grp, qn)

    qn = lax.fori_loop(0, E // ECHUNK, scan_chunk, jnp.int32(0))
    ngrp = (qn + 15) // 16

    def channel(s, _):
        pltpu.sync_copy(asp.at[pl.ds(s * NP, NP)], asv)
        pltpu.sync_copy(adp.at[pl.ds(s * NP + lo, NODES_W)], adw)

        def zd(i, _):
            denw[pl.ds(i * 16, 16)] = jnp.zeros((16,), jnp.float32)
            return 0
        lax.fori_loop(0, NODES_W // 16, zd, 0)

        def za(i, _):
            acc[pl.ds(i * 16, 16)] = jnp.full((16,), NEG, jnp.float32)
            return 0
        lax.fori_loop(0, (NODES_W + 1) * HPAD // 16, za, 0)

        def p1(g, _):
            base = g * 16
            m = (base + lane) < qn
            vsrc = qsrc[pl.ds(base, 16)]
            vdl = qdstl[pl.ds(base, 16)]
            va = plsc.load_gather(asv, [vsrc])
            vb = plsc.load_gather(adw, [vdl])
            ve = va + vb
            ve = jnp.maximum(ve, 0.2 * ve)
            vex = jnp.exp(ve)
            qex[pl.ds(base, 16)] = vex
            plsc.addupdate_scatter(denw, [vdl], jnp.where(m, vex, 0.0))
            return 0
        lax.fori_loop(0, ngrp, p1, 0)

        def p2(g, _):
            base = g * 16
            vdl = qdstl[pl.ds(base, 16)]
            vden = plsc.load_gather(denw, [vdl])
            qex[pl.ds(base, 16)] = qex[pl.ds(base, 16)] / (vden + 1e-16)
            return 0
        lax.fori_loop(0, ngrp, p2, 0)

        nblk = (qn + GB - 1) // GB

        def blk(b, _):
            qb = b * GB

            def bld(g, _):
                v = qsrc[pl.ds(qb + g * 16, 16)] * 5 + s
                qrow[pl.ds(g * 16, 16)] = v
                return 0
            lax.fori_loop(0, GB // 16, bld, 0)
            pltpu.async_copy(tbl.at[qrow], rows, sem).wait()

            def grp16(gi, _):
                base = qb + gi * 16
                valid = (base + lane) < qn
                alpha16 = qex[pl.ds(base, 16)]
                dl16 = jnp.where(valid, qdstl[pl.ds(base, 16)], NODES_W)
                for i in range(16):
                    a = alpha16[i]
                    ab = dl16[i] * HPAD
                    jj = gi * 16 + i
                    for t in range(3):
                        rv = rows[jj, pl.ds(t * 16, 16)]
                        av = acc[pl.ds(ab + t * 16, 16)]
                        acc[pl.ds(ab + t * 16, 16)] = jnp.maximum(av, a * rv)
                return 0
            lax.fori_loop(0, GB // 16, grp16, 0)
            return 0
        lax.fori_loop(0, nblk, blk, 0)

        def ep(r, _):
            rb = r * HPAD
            for t in range(3):
                a = acc[pl.ds(rb + t * 16, 16)]
                bv = biasv[pl.ds(t * 16, 16)]
                v = jnp.where(a < -1.0e37, 0.0, a) + bv
                acc[pl.ds(rb + t * 16, 16)] = jnp.maximum(v, 0.0)
            return 0
        lax.fori_loop(0, NODES_W, ep, 0)

        def rp(g, _):
            x = g * 16 + lane
            q = (x * 52429) >> 21
            cidx = x - 40 * q
            accb[pl.ds(g * 16, 16)] = plsc.load_gather(acc, [q * HPAD + cidx])
            return 0
        lax.fori_loop(0, NODES_W * HIDDEN // 16, rp, 0)
        base = (s * N_NODES + lo) * HIDDEN

        @pl.when(lo + NODES_W <= N_NODES)
        def _():
            pltpu.sync_copy(accb.at[pl.ds(0, NODES_W * HIDDEN)],
                            out.at[pl.ds(base, NODES_W * HIDDEN)])

        @pl.when(lo + NODES_W > N_NODES)
        def _():
            pltpu.sync_copy(accb.at[pl.ds(0, 80 * HIDDEN)],
                            out.at[pl.ds(base, 80 * HIDDEN)])
        return 0

    lax.fori_loop(0, 5, channel, 0)


def _sc_edge_stage(tbl, srcv, dstv, asp, adp, bias48):
    mesh = plsc.VectorSubcoreMesh(core_axis_name="c", subcore_axis_name="s")
    f = functools.partial(
        pl.kernel,
        mesh=mesh,
        compiler_params=pltpu.CompilerParams(needs_layout_passes=False,
                                             use_tc_tiling_on_sc=False),
        out_type=jax.ShapeDtypeStruct((5 * N_NODES * HIDDEN,), jnp.float32),
        scratch_types=[
            pltpu.VMEM((QCAP,), jnp.int32),
            pltpu.VMEM((QCAP,), jnp.int32),
            pltpu.VMEM((QCAP,), jnp.float32),
            pltpu.VMEM((NP,), jnp.float32),
            pltpu.VMEM((NODES_W,), jnp.float32),
            pltpu.VMEM((NODES_W,), jnp.float32),
            pltpu.VMEM(((NODES_W + 1) * HPAD,), jnp.float32),
            pltpu.VMEM((NODES_W * HIDDEN,), jnp.float32),
            pltpu.VMEM((ECHUNK,), jnp.int32),
            pltpu.VMEM((ECHUNK,), jnp.int32),
            pltpu.VMEM((GB,), jnp.int32),
            pltpu.VMEM((GB, HPAD), jnp.float32),
            pltpu.VMEM((HPAD,), jnp.float32),
            pltpu.SemaphoreType.DMA,
        ],
    )(_sc_edge_body)
    return f(tbl, srcv, dstv, asp, adp, bias48)


def kernel(x, edge_index, conv_w, conv_b, lin_w, lin_b, gat_w, att_src,
           att_dst, gat_bias):
    n = x.shape[0]
    xs2 = x[:, :, 3:].reshape(n * 5, 392)
    w_even, w_odd = _make_band_weights(conv_w)
    we = jnp.stack([jax.lax.dynamic_slice(w_even, (96 * k, 96 * k), (CW, PC))
                    for k in range(NCHUNK)])
    wo = jnp.stack([jax.lax.dynamic_slice(w_odd, (96 * k, 96 * k), (CW, PC))
                    for k in range(NCHUNK)])
    cb_cols = jnp.tile(conv_b, PC // 8)
    q = jnp.arange(48)
    c = jnp.arange(8)
    rows = (c[None, :] * 48 + q[:, None]).reshape(-1)
    lw_perm = lin_w[rows].reshape(NCHUNK, PC, 80)

    out48 = _cnn_stage(xs2, we, wo, cb_cols, lw_perm, lin_b, gat_w,
                       att_src, att_dst)

    src = edge_index[0].astype(jnp.int32)
    dst = edge_index[1].astype(jnp.int32)
    asp = jnp.pad(out48[:, HIDDEN].reshape(n, 5).T, ((0, 0), (0, NP - n))).reshape(-1)
    adp = jnp.pad(out48[:, HIDDEN + 1].reshape(n, 5).T, ((0, 0), (0, NP - n))).reshape(-1)
    bias48 = jnp.pad(gat_bias, (0, HPAD - HIDDEN))

    outp = _sc_edge_stage(out48, src, dst, asp, adp, bias48)
    return outp.reshape(5, n, HIDDEN)


# speedup vs baseline: 24.5897x; 1.0918x over previous
import functools

import jax
import jax.numpy as jnp
from jax import lax
from jax.experimental import pallas as pl
from jax.experimental.pallas import tpu as pltpu
from jax.experimental.pallas import tpu_sc as plsc

N_NODES = 10000
HIDDEN = 40
HPAD = 48
NP = 10240
NODES_W = 320
NW = 32
E = 640000
ECHUNK = 4000
QCAP = 23040
GB = 128
NEG = -3.0e38
NCHUNK = 4
CW = 104
PC = 96


def _cnn_body(x_ref, we_ref, wo_ref, cb_ref, lw_ref, lb_ref, gw_ref,
              asv_ref, adv_ref, out_ref):
    M = x_ref.shape[0]
    acc = jnp.broadcast_to(lb_ref[...][None, :], (M, 80))
    for k in range(NCHUNK):
        xk = x_ref[:, 96 * k: 96 * k + CW]
        ye = xk @ we_ref[k] + cb_ref[...][None, :]
        yo = xk @ wo_ref[k] + cb_ref[...][None, :]
        zk = jnp.maximum(jnp.maximum(ye, yo), 0.0)
        acc = acc + zk @ lw_ref[k]
    feat = jnp.maximum(acc, 0.0)
    hl = feat @ gw_ref[...]
    a_s = jnp.sum(hl * asv_ref[...][None, :], axis=-1, keepdims=True)
    a_d = jnp.sum(hl * adv_ref[...][None, :], axis=-1, keepdims=True)
    out_ref[:, 0:HIDDEN] = hl
    out_ref[:, HIDDEN:HIDDEN + 1] = a_s
    out_ref[:, HIDDEN + 1:HIDDEN + 2] = a_d
    out_ref[:, HIDDEN + 2:HPAD] = jnp.zeros((M, HPAD - HIDDEN - 2),
                                            dtype=jnp.float32)


def _make_band_weights(conv_w):
    w = conv_w[:, 0, :]
    i = jnp.arange(392)[:, None]
    q = jnp.arange(48)[None, :]
    cols_c = jnp.arange(8)[None, :]

    def band(off):
        j = i - 8 * q - off
        valid = (j >= 0) & (j < 8)
        jc = jnp.clip(j, 0, 7)
        taps = w.T[jc]
        taps = jnp.where(valid[:, :, None], taps, 0.0)
        return taps.reshape(392, 48 * 8)

    return band(0), band(4)


def _cnn_stage(xs2, we, wo, cb_cols, lw_perm, lin_b, gat_w, att_src, att_dst):
    M = xs2.shape[0]
    B = 2000
    grid = (M // B,)
    full = lambda *s: pl.BlockSpec(s, lambda i: tuple(0 for _ in s))
    return pl.pallas_call(
        _cnn_body,
        grid=grid,
        in_specs=[
            pl.BlockSpec((B, 392), lambda i: (i, 0)),
            full(NCHUNK, CW, PC), full(NCHUNK, CW, PC), full(PC,),
            full(NCHUNK, PC, 80), full(80,), full(80, HIDDEN),
            full(HIDDEN,), full(HIDDEN,),
        ],
        out_specs=pl.BlockSpec((B, HPAD), lambda i: (i, 0)),
        out_shape=jax.ShapeDtypeStruct((M, HPAD), jnp.float32),
    )(xs2, we, wo, cb_cols, lw_perm, lin_b, gat_w, att_src, att_dst)


def _sc_edge_body(tbl, srcv, dstv, asp, adp, bias48, out,
                  qsrc, qdstl, qex, asv, adw, denw, acc, accb, srcc, dstc,
                  qrow, rows, biasv, sem, sem2):
    wid = lax.axis_index("s") * 2 + lax.axis_index("c")
    lo = wid * NODES_W
    lane = lax.broadcasted_iota(jnp.int32, (16,), 0)

    pltpu.sync_copy(bias48, biasv)

    def zi(i, _):
        z = jnp.zeros((16,), jnp.int32)
        qsrc[pl.ds(i * 16, 16)] = z
        qdstl[pl.ds(i * 16, 16)] = z
        return 0
    lax.fori_loop(0, QCAP // 16, zi, 0)

    def scan_chunk(ci, qn):
        off = ci * ECHUNK
        pltpu.sync_copy(srcv.at[pl.ds(off, ECHUNK)], srcc)
        pltpu.sync_copy(dstv.at[pl.ds(off, ECHUNK)], dstc)

        def grp(g, qn):
            vs = srcc[pl.ds(g * 16, 16)]
            vd = dstc[pl.ds(g * 16, 16)]
            vdl = vd - lo
            inb = vdl.astype(jnp.uint32) < jnp.uint32(NODES_W)
            plsc.store_compressed(qsrc.at[pl.ds(qn, 16)], vs, mask=inb)
            plsc.store_compressed(qdstl.at[pl.ds(qn, 16)], vdl, mask=inb)
            return qn + jnp.sum(inb.astype(jnp.int32))

        return lax.fori_loop(0, ECHUNK // 16, grp, qn)

    qn = lax.fori_loop(0, E // ECHUNK, scan_chunk, jnp.int32(0))
    ngrp = (qn + 15) // 16

    def channel(s, _):
        pltpu.sync_copy(asp.at[pl.ds(s * NP, NP)], asv)
        pltpu.sync_copy(adp.at[pl.ds(s * NP + lo, NODES_W)], adw)

        def zd(i, _):
            denw[pl.ds(i * 16, 16)] = jnp.zeros((16,), jnp.float32)
            return 0
        lax.fori_loop(0, NODES_W // 16, zd, 0)

        def za(i, _):
            acc[pl.ds(i * 16, 16)] = jnp.full((16,), NEG, jnp.float32)
            return 0
        lax.fori_loop(0, (NODES_W + 1) * HPAD // 16, za, 0)

        def p1(g, _):
            base = g * 16
            m = (base + lane) < qn
            vsrc = qsrc[pl.ds(base, 16)]
            vdl = qdstl[pl.ds(base, 16)]
            va = plsc.load_gather(asv, [vsrc])
            vb = plsc.load_gather(adw, [vdl])
            ve = va + vb
            ve = jnp.maximum(ve, 0.2 * ve)
            vex = jnp.exp(ve)
            qex[pl.ds(base, 16)] = vex
            plsc.addupdate_scatter(denw, [vdl], jnp.where(m, vex, 0.0))
            return 0
        lax.fori_loop(0, ngrp, p1, 0)

        def p2(g, _):
            base = g * 16
            vdl = qdstl[pl.ds(base, 16)]
            vden = plsc.load_gather(denw, [vdl])
            qex[pl.ds(base, 16)] = qex[pl.ds(base, 16)] / (vden + 1e-16)
            return 0
        lax.fori_loop(0, ngrp, p2, 0)

        nblk = (qn + GB - 1) // GB

        def bld(half, b):
            def bl(g, _):
                v = qsrc[pl.ds(b * GB + g * 16, 16)] * 5 + s
                qrow[half, pl.ds(g * 16, 16)] = v
                return 0
            lax.fori_loop(0, GB // 16, bl, 0)

        def proc(half, b):
            qb = b * GB

            def grp16(gi, _):
                base = qb + gi * 16
                valid = (base + lane) < qn
                alpha16 = qex[pl.ds(base, 16)]
                dl16 = jnp.where(valid, qdstl[pl.ds(base, 16)], NODES_W)
                for i in range(16):
                    a = alpha16[i]
                    ab = dl16[i] * HPAD
                    jj = gi * 16 + i
                    for t in range(3):
                        rv = rows[half, jj, pl.ds(t * 16, 16)]
                        av = acc[pl.ds(ab + t * 16, 16)]
                        acc[pl.ds(ab + t * 16, 16)] = jnp.maximum(av, a * rv)
                return 0
            lax.fori_loop(0, GB // 16, grp16, 0)

        def blk2(b2, _):
            b0 = b2 * 2
            b1 = b0 + 1
            bld(0, b0)
            cp0 = pltpu.async_copy(tbl.at[qrow.at[0]], rows.at[0], sem)

            @pl.when(b1 < nblk)
            def _():
                bld(1, b1)
                pltpu.async_copy(tbl.at[qrow.at[1]], rows.at[1], sem2)

            cp0.wait()
            proc(0, b0)

            @pl.when(b1 < nblk)
            def _():
                pltpu.make_async_copy(tbl.at[qrow.at[1]], rows.at[1],
                                      sem2).wait()
                proc(1, b1)
            return 0
        lax.fori_loop(0, (nblk + 1) // 2, blk2, 0)

        def ep(r, _):
            rb = r * HPAD
            for t in range(3):
                a = acc[pl.ds(rb + t * 16, 16)]
                bv = biasv[pl.ds(t * 16, 16)]
                v = jnp.where(a < -1.0e37, 0.0, a) + bv
                acc[pl.ds(rb + t * 16, 16)] = jnp.maximum(v, 0.0)
            return 0
        lax.fori_loop(0, NODES_W, ep, 0)

        def rp(g, _):
            x = g * 16 + lane
            q = (x * 52429) >> 21
            cidx = x - 40 * q
            accb[pl.ds(g * 16, 16)] = plsc.load_gather(acc, [q * HPAD + cidx])
            return 0
        lax.fori_loop(0, NODES_W * HIDDEN // 16, rp, 0)
        base = (s * N_NODES + lo) * HIDDEN

        @pl.when(lo + NODES_W <= N_NODES)
        def _():
            pltpu.sync_copy(accb.at[pl.ds(0, NODES_W * HIDDEN)],
                            out.at[pl.ds(base, NODES_W * HIDDEN)])

        @pl.when(lo + NODES_W > N_NODES)
        def _():
            pltpu.sync_copy(accb.at[pl.ds(0, 80 * HIDDEN)],
                            out.at[pl.ds(base, 80 * HIDDEN)])
        return 0

    lax.fori_loop(0, 5, channel, 0)


def _sc_edge_stage(tbl, srcv, dstv, asp, adp, bias48):
    mesh = plsc.VectorSubcoreMesh(core_axis_name="c", subcore_axis_name="s")
    f = functools.partial(
        pl.kernel,
        mesh=mesh,
        compiler_params=pltpu.CompilerParams(needs_layout_passes=False,
                                             use_tc_tiling_on_sc=False),
        out_type=jax.ShapeDtypeStruct((5 * N_NODES * HIDDEN,), jnp.float32),
        scratch_types=[
            pltpu.VMEM((QCAP,), jnp.int32),
            pltpu.VMEM((QCAP,), jnp.int32),
            pltpu.VMEM((QCAP,), jnp.float32),
            pltpu.VMEM((NP,), jnp.float32),
            pltpu.VMEM((NODES_W,), jnp.float32),
            pltpu.VMEM((NODES_W,), jnp.float32),
            pltpu.VMEM(((NODES_W + 1) * HPAD,), jnp.float32),
            pltpu.VMEM((NODES_W * HIDDEN,), jnp.float32),
            pltpu.VMEM((ECHUNK,), jnp.int32),
            pltpu.VMEM((ECHUNK,), jnp.int32),
            pltpu.VMEM((2, GB), jnp.int32),
            pltpu.VMEM((2, GB, HPAD), jnp.float32),
            pltpu.VMEM((HPAD,), jnp.float32),
            pltpu.SemaphoreType.DMA,
            pltpu.SemaphoreType.DMA,
        ],
    )(_sc_edge_body)
    return f(tbl, srcv, dstv, asp, adp, bias48)


def kernel(x, edge_index, conv_w, conv_b, lin_w, lin_b, gat_w, att_src,
           att_dst, gat_bias):
    n = x.shape[0]
    xs2 = x[:, :, 3:].reshape(n * 5, 392)
    w_even, w_odd = _make_band_weights(conv_w)
    we = jnp.stack([jax.lax.dynamic_slice(w_even, (96 * k, 96 * k), (CW, PC))
                    for k in range(NCHUNK)])
    wo = jnp.stack([jax.lax.dynamic_slice(w_odd, (96 * k, 96 * k), (CW, PC))
                    for k in range(NCHUNK)])
    cb_cols = jnp.tile(conv_b, PC // 8)
    q = jnp.arange(48)
    c = jnp.arange(8)
    rows = (c[None, :] * 48 + q[:, None]).reshape(-1)
    lw_perm = lin_w[rows].reshape(NCHUNK, PC, 80)

    out48 = _cnn_stage(xs2, we, wo, cb_cols, lw_perm, lin_b, gat_w,
                       att_src, att_dst)

    src = edge_index[0].astype(jnp.int32)
    dst = edge_index[1].astype(jnp.int32)
    asp = jnp.pad(out48[:, HIDDEN].reshape(n, 5).T, ((0, 0), (0, NP - n))).reshape(-1)
    adp = jnp.pad(out48[:, HIDDEN + 1].reshape(n, 5).T, ((0, 0), (0, NP - n))).reshape(-1)
    bias48 = jnp.pad(gat_bias, (0, HPAD - HIDDEN))

    outp = _sc_edge_stage(out48, src, dst, asp, adp, bias48)
    return outp.reshape(5, n, HIDDEN)
